# Initial kernel scaffold; baseline (speedup 1.0000x reference)
import jax
import jax.numpy as jnp
from jax.experimental import pallas as pl


def kernel(x_q, user_static, short_term, long_term, W_ctx, norm_k_w, norm_v_w, norm_qkv_w, Wqkv, Wo_self, norm_q_w, Wq, Wo_cross, moe_norm_w, Wgate, We1, be1, We2, be2, final_norm_w):
    raise NotImplementedError("write your pallas kernel here")


# speedup vs baseline: 1.4058x; 1.4058x over previous
import functools
import math

import jax
import jax.numpy as jnp
from jax.experimental import pallas as pl
from jax.experimental.pallas import tpu as pltpu

L = 2
D = 768
HQ = 12
GKV = 4
DH = 64
E = 8
DFF = 1536
SKV = 2
EPS = 1e-6
TQ = 2048
TC = 251
TCP = 256
MID = GKV * DH
CHUNK = SKV * MID
REP = HQ // GKV
TB = 256
NB = TQ // TB
TBM = 256
NBM = TQ // TBM
INV_SQRT_DH = 1.0 / math.sqrt(DH)


def _rms(x, w):
    return x * jax.lax.rsqrt(jnp.mean(x * x, axis=-1, keepdims=True) + EPS) * w


def _dot(a, b):
    return jnp.dot(a, b, preferred_element_type=jnp.float32)


def _ctx_kernel(xc_ref, wctx_ref, nkw_ref, nvw_ref, kc_ref, vc_ref):
    ctx = _dot(xc_ref[...], wctx_ref[...])
    for l in range(L):
        ch = ctx[:, l * CHUNK:(l + 1) * CHUNK]
        kc_ref[l] = _rms(ch[:, :MID], nkw_ref[l])
        vc_ref[l] = _rms(ch[:, MID:], nvw_ref[l])


def _ctx_kv(xc, W_ctx, norm_k_w, norm_v_w):
    return pl.pallas_call(
        _ctx_kernel,
        out_shape=(
            jax.ShapeDtypeStruct((L, TCP, MID), jnp.float32),
            jax.ShapeDtypeStruct((L, TCP, MID), jnp.float32),
        ),
    )(xc, W_ctx, norm_k_w, norm_v_w)


def _qkv_kernel(x_ref, nw_ref, w_ref, q_ref, k_ref, v_ref):
    xn = _rms(x_ref[...], nw_ref[...])
    qkv = _dot(xn, w_ref[...])
    q_ref[...] = qkv[:, :D]
    k_ref[...] = qkv[:, D:2 * D]
    v_ref[...] = qkv[:, 2 * D:]


def _qkv_proj(x, nw, Wqkv_l):
    return pl.pallas_call(
        _qkv_kernel,
        grid=(NB,),
        in_specs=[
            pl.BlockSpec((TB, D), lambda i: (i, 0)),
            pl.BlockSpec((1, D), lambda i: (0, 0)),
            pl.BlockSpec((D, 3 * D), lambda i: (0, 0)),
        ],
        out_specs=(
            pl.BlockSpec((TB, D), lambda i: (i, 0)),
            pl.BlockSpec((TB, D), lambda i: (i, 0)),
            pl.BlockSpec((TB, D), lambda i: (i, 0)),
        ),
        out_shape=(
            jax.ShapeDtypeStruct((TQ, D), jnp.float32),
            jax.ShapeDtypeStruct((TQ, D), jnp.float32),
            jax.ShapeDtypeStruct((TQ, D), jnp.float32),
        ),
    )(x, nw, Wqkv_l)


def _self_attn_kernel(q_ref, k_ref, v_ref, o_ref):
    i = pl.program_id(0)
    rows = i * TB + jax.lax.broadcasted_iota(jnp.int32, (TB, TQ), 0)
    cols = jax.lax.broadcasted_iota(jnp.int32, (TB, TQ), 1)
    mask = jnp.where(cols > rows, -1e9, 0.0)
    outs = []
    for h in range(HQ):
        qh = q_ref[:, h * DH:(h + 1) * DH] * INV_SQRT_DH
        kh = k_ref[:, h * DH:(h + 1) * DH]
        s = jax.lax.dot_general(qh, kh, (((1,), (1,)), ((), ())),
                                preferred_element_type=jnp.float32)
        s = s + mask
        m = jnp.max(s, axis=-1, keepdims=True)
        p = jnp.exp(s - m)
        p = p / jnp.sum(p, axis=-1, keepdims=True)
        outs.append(_dot(p, v_ref[:, h * DH:(h + 1) * DH]))
    o_ref[...] = jnp.concatenate(outs, axis=1)


def _self_attn(q, k, v):
    return pl.pallas_call(
        _self_attn_kernel,
        grid=(NB,),
        in_specs=[
            pl.BlockSpec((TB, D), lambda i: (i, 0)),
            pl.BlockSpec((TQ, D), lambda i: (0, 0)),
            pl.BlockSpec((TQ, D), lambda i: (0, 0)),
        ],
        out_specs=pl.BlockSpec((TB, D), lambda i: (i, 0)),
        out_shape=jax.ShapeDtypeStruct((TQ, D), jnp.float32),
    )(q, k, v)


def _proj_q_kernel(a_ref, x_ref, wo_ref, nqw_ref, wq_ref, x1_ref, q2_ref):
    x1 = x_ref[...] + _dot(a_ref[...], wo_ref[...])
    x1_ref[...] = x1
    xn = _rms(x1, nqw_ref[...])
    q2_ref[...] = _dot(xn, wq_ref[...])


def _proj_q(attn_out, x, Wo_l, nqw, Wq_l):
    return pl.pallas_call(
        _proj_q_kernel,
        grid=(NB,),
        in_specs=[
            pl.BlockSpec((TB, D), lambda i: (i, 0)),
            pl.BlockSpec((TB, D), lambda i: (i, 0)),
            pl.BlockSpec((D, D), lambda i: (0, 0)),
            pl.BlockSpec((1, D), lambda i: (0, 0)),
            pl.BlockSpec((D, D), lambda i: (0, 0)),
        ],
        out_specs=(
            pl.BlockSpec((TB, D), lambda i: (i, 0)),
            pl.BlockSpec((TB, D), lambda i: (i, 0)),
        ),
        out_shape=(
            jax.ShapeDtypeStruct((TQ, D), jnp.float32),
            jax.ShapeDtypeStruct((TQ, D), jnp.float32),
        ),
    )(attn_out, x, Wo_l, nqw, Wq_l)


def _cross_gate_kernel(q_ref, x1_ref, kc_ref, vc_ref, woc_ref, mnw_ref, wg_ref,
                       x2_ref, xn3_ref, eid_ref, top1_ref):
    kmask = jnp.where(
        jax.lax.broadcasted_iota(jnp.int32, (TB, TCP), 1) >= TC, -1e30, 0.0)
    outs = []
    for h in range(HQ):
        g = h // REP
        qh = q_ref[:, h * DH:(h + 1) * DH] * INV_SQRT_DH
        kh = kc_ref[:, g * DH:(g + 1) * DH]
        s = jax.lax.dot_general(qh, kh, (((1,), (1,)), ((), ())),
                                preferred_element_type=jnp.float32)
        s = s + kmask
        m = jnp.max(s, axis=-1, keepdims=True)
        p = jnp.exp(s - m)
        p = p / jnp.sum(p, axis=-1, keepdims=True)
        outs.append(_dot(p, vc_ref[:, g * DH:(g + 1) * DH]))
    co = jnp.concatenate(outs, axis=1)
    x2 = x1_ref[...] + _dot(co, woc_ref[...])
    x2_ref[...] = x2
    xn3 = _rms(x2, mnw_ref[...])
    xn3_ref[...] = xn3
    glog = _dot(xn3, wg_ref[...])
    m = jnp.max(glog, axis=-1, keepdims=True)
    gp = jnp.exp(glog - m)
    gs = gp / jnp.sum(gp, axis=-1, keepdims=True)
    eid_ref[0, 0] = jnp.argmax(gs, axis=-1).astype(jnp.int32)
    top1_ref[0, 0] = jnp.max(gs, axis=-1)


def _cross_gate(q2, x1, kc_l, vc_l, Woc_l, mnw, Wg_l):
    return pl.pallas_call(
        _cross_gate_kernel,
        grid=(NB,),
        in_specs=[
            pl.BlockSpec((TB, D), lambda i: (i, 0)),
            pl.BlockSpec((TB, D), lambda i: (i, 0)),
            pl.BlockSpec((TCP, MID), lambda i: (0, 0)),
            pl.BlockSpec((TCP, MID), lambda i: (0, 0)),
            pl.BlockSpec((D, D), lambda i: (0, 0)),
            pl.BlockSpec((1, D), lambda i: (0, 0)),
            pl.BlockSpec((D, E), lambda i: (0, 0)),
        ],
        out_specs=(
            pl.BlockSpec((TB, D), lambda i: (i, 0)),
            pl.BlockSpec((TB, D), lambda i: (i, 0)),
            pl.BlockSpec((1, 1, TB), lambda i: (i, 0, 0)),
            pl.BlockSpec((1, 1, TB), lambda i: (i, 0, 0)),
        ),
        out_shape=(
            jax.ShapeDtypeStruct((TQ, D), jnp.float32),
            jax.ShapeDtypeStruct((TQ, D), jnp.float32),
            jax.ShapeDtypeStruct((NB, 1, TB), jnp.int32),
            jax.ShapeDtypeStruct((NB, 1, TB), jnp.float32),
        ),
    )(q2, x1, kc_l, vc_l, Woc_l, mnw, Wg_l)


def _moe_kernel(bounds_ref, xs_ref, we1_ref, be1_ref, we2_ref, be2_ref, out_ref):
    e = pl.program_id(0)
    start = bounds_ref[e]
    end = bounds_ref[E + e]

    @pl.when(e == 0)
    def _():
        out_ref[...] = jnp.zeros_like(out_ref)

    for b in range(NBM):
        r0 = b * TBM

        @pl.when((start < r0 + TBM) & (end > r0))
        def _():
            xb = xs_ref[r0:r0 + TBM, :]
            h = _dot(xb, we1_ref[0]) + be1_ref[0]
            h = h * jax.nn.sigmoid(h)
            y = _dot(h, we2_ref[0]) + be2_ref[0]
            rows = r0 + jax.lax.broadcasted_iota(jnp.int32, (TBM, D), 0)
            keep = (rows >= start) & (rows < end)
            out_ref[r0:r0 + TBM, :] += jnp.where(keep, y, 0.0)


def _moe_ffn(xs, bounds, We1_l, be1_l, We2_l, be2_l):
    grid_spec = pltpu.PrefetchScalarGridSpec(
        num_scalar_prefetch=1,
        grid=(E,),
        in_specs=[
            pl.BlockSpec((TQ, D), lambda e, b: (0, 0)),
            pl.BlockSpec((1, D, DFF), lambda e, b: (e, 0, 0)),
            pl.BlockSpec((1, 1, DFF), lambda e, b: (e, 0, 0)),
            pl.BlockSpec((1, DFF, D), lambda e, b: (e, 0, 0)),
            pl.BlockSpec((1, 1, D), lambda e, b: (e, 0, 0)),
        ],
        out_specs=pl.BlockSpec((TQ, D), lambda e, b: (0, 0)),
    )
    return pl.pallas_call(
        _moe_kernel,
        grid_spec=grid_spec,
        out_shape=jax.ShapeDtypeStruct((TQ, D), jnp.float32),
    )(bounds, xs, We1_l, be1_l, We2_l, be2_l)


def _combine_kernel_plain(x2_ref, y_ref, t1_ref, o_ref):
    o_ref[...] = x2_ref[...] + y_ref[...] * t1_ref[0, 0][:, None]


def _combine_kernel_final(x2_ref, y_ref, t1_ref, fw_ref, o_ref):
    x3 = x2_ref[...] + y_ref[...] * t1_ref[0, 0][:, None]
    o_ref[...] = _rms(x3, fw_ref[...])


def _combine(x2, yu, top1, final_w=None):
    in_specs = [
        pl.BlockSpec((TB, D), lambda i: (i, 0)),
        pl.BlockSpec((TB, D), lambda i: (i, 0)),
        pl.BlockSpec((1, 1, TB), lambda i: (i, 0, 0)),
    ]
    args = [x2, yu, top1]
    if final_w is None:
        body = _combine_kernel_plain
    else:
        body = _combine_kernel_final
        in_specs.append(pl.BlockSpec((1, D), lambda i: (0, 0)))
        args.append(final_w)
    return pl.pallas_call(
        body,
        grid=(NB,),
        in_specs=in_specs,
        out_specs=pl.BlockSpec((TB, D), lambda i: (i, 0)),
        out_shape=jax.ShapeDtypeStruct((TQ, D), jnp.float32),
    )(*args)


def kernel(x_q, user_static, short_term, long_term, W_ctx, norm_k_w, norm_v_w,
           norm_qkv_w, Wqkv, Wo_self, norm_q_w, Wq, Wo_cross, moe_norm_w,
           Wgate, We1, be1, We2, be2, final_norm_w):
    x = x_q[0]
    xc = jnp.concatenate(
        [user_static[0], short_term[0], long_term[0],
         jnp.zeros((TCP - TC, D), jnp.float32)], axis=0)
    kc, vc = _ctx_kv(xc, W_ctx, norm_k_w, norm_v_w)

    be1r = be1.reshape(L, E, 1, DFF)
    be2r = be2.reshape(L, E, 1, D)

    for l in range(L):
        q, k, v = _qkv_proj(x, norm_qkv_w[l][None, :], Wqkv[l])
        attn = _self_attn(q, k, v)
        x1, q2 = _proj_q(attn, x, Wo_self[l], norm_q_w[l][None, :], Wq[l])
        x2, xn3, eid, top1 = _cross_gate(
            q2, x1, kc[l], vc[l], Wo_cross[l], moe_norm_w[l][None, :], Wgate[l])

        eid_flat = eid.reshape(TQ)
        order = jnp.argsort(eid_flat)
        sorted_eid = eid_flat[order]
        xs = xn3[order]
        starts = jnp.searchsorted(sorted_eid, jnp.arange(E, dtype=jnp.int32),
                                  side='left')
        ends = jnp.searchsorted(sorted_eid, jnp.arange(E, dtype=jnp.int32),
                                side='right')
        bounds = jnp.concatenate([starts, ends]).astype(jnp.int32)
        ys = _moe_ffn(xs, bounds, We1[l], be1r[l], We2[l], be2r[l])
        inv = jnp.argsort(order)
        yu = ys[inv]
        fw = final_norm_w[None, :] if l == L - 1 else None
        x = _combine(x2, yu, top1, fw)

    return x[None]
